# baseline (speedup 1.0000x reference)
import functools

import jax
import jax.numpy as jnp
from jax import lax
from jax.experimental import pallas as pl
from jax.experimental.pallas import tpu as pltpu
from jax.experimental.pallas import tpu_sc as plsc

N = 10000
NP = 10240
E = 320000
D = 128
DE = 16
F = 256
NC = 2
NS = 16
NW = NC * NS
EPW = E // NW
RPT = NP // NS

CH1, SBR1 = 40, 5
CH1B, SBR1B = 40, 5
CH2, SBR2 = 40, 5
CH3, SBR3 = 40, 5


def _mesh():
    return plsc.VectorSubcoreMesh(core_axis_name="c", subcore_axis_name="s")


def _sc_pass1a_body(atom_hbm, sidx_hbm, nidx_hbm, z128_hbm, v_out,
                    v_acc, idxs_v, idxn_v, rows_v, sem):
    c = lax.axis_index("c")
    s = lax.axis_index("s")
    wid = s * NC + c
    nsb = EPW // (CH1 * SBR1)

    row0 = s * RPT
    pltpu.sync_copy(z128_hbm.at[pl.ds(row0, RPT)], v_acc.at[pl.ds(row0, RPT)])
    plsc.subcore_barrier()

    def body(sb, carry):
        pltpu.sync_copy(sidx_hbm.at[wid, sb], idxs_v)
        pltpu.sync_copy(nidx_hbm.at[wid, sb], idxn_v)
        gets = [
            pltpu.async_copy(atom_hbm.at[idxn_v.at[j]],
                             rows_v.at[pl.ds(j * CH1, CH1)], sem)
            for j in range(SBR1)
        ]
        for g in gets:
            g.wait()
        for j in range(SBR1):
            pltpu.sync_copy(rows_v.at[pl.ds(j * CH1, CH1)],
                            v_acc.at[idxs_v.at[j]], add=True)
        return carry

    lax.fori_loop(0, nsb, body, 0)
    plsc.subcore_barrier()
    sl = pl.ds(row0, RPT)
    pltpu.sync_copy(v_acc.at[sl], v_out.at[c, sl])


def _sc_pass1a(atom, sidx4, nidx4, z128):
    kfn = pl.kernel(
        _sc_pass1a_body,
        out_type=jax.ShapeDtypeStruct((NC, NP, D), jnp.float32),
        mesh=_mesh(),
        scratch_types=[
            pltpu.VMEM_SHARED((NP, D), jnp.float32),
            pltpu.VMEM((SBR1, CH1), jnp.int32),
            pltpu.VMEM((SBR1, CH1), jnp.int32),
            pltpu.VMEM((SBR1 * CH1, D), jnp.float32),
            pltpu.SemaphoreType.DMA,
        ],
    )
    return kfn(atom, sidx4, nidx4, z128)


def _sc_pass1b_body(nbr128_hbm, idx_hbm, z128_hbm, u_out,
                    u_acc, idx_v, wide_v, sem):
    c = lax.axis_index("c")
    s = lax.axis_index("s")
    wid = s * NC + c
    sbe = CH1B * SBR1B
    nsb = EPW // sbe

    row0 = s * RPT
    pltpu.sync_copy(z128_hbm.at[pl.ds(row0, RPT)], u_acc.at[pl.ds(row0, RPT)])
    plsc.subcore_barrier()

    def body(sb, carry):
        ebase = wid * EPW + sb * sbe
        pltpu.sync_copy(idx_hbm.at[wid, sb], idx_v)
        g = pltpu.async_copy(nbr128_hbm.at[pl.ds(ebase, sbe)], wide_v, sem)
        g.wait()
        for j in range(SBR1B):
            sl = pl.ds(j * CH1B, CH1B)
            pltpu.sync_copy(wide_v.at[sl], u_acc.at[idx_v.at[j]], add=True)
        return carry

    lax.fori_loop(0, nsb, body, 0)
    plsc.subcore_barrier()
    sl = pl.ds(row0, RPT)
    pltpu.sync_copy(u_acc.at[sl], u_out.at[c, sl])


def _sc_pass1b(nbr128, idx4, z128):
    kfn = pl.kernel(
        _sc_pass1b_body,
        out_type=jax.ShapeDtypeStruct((NC, NP, D), jnp.float32),
        mesh=_mesh(),
        scratch_types=[
            pltpu.VMEM_SHARED((NP, D), jnp.float32),
            pltpu.VMEM((SBR1B, CH1B), jnp.int32),
            pltpu.VMEM((SBR1B * CH1B, D), jnp.float32),
            pltpu.SemaphoreType.DMA,
        ],
    )
    return kfn(nbr128, idx4, z128)


_SBE2 = CH2 * SBR2
_NSB2 = EPW // _SBE2


def _sc_pass2a_body(ps_hbm, pn_hbm, sidx_hbm, nidx_hbm, zs_out, zn_out,
                    is0, in0, is1, in1, zs0, zn0, zs1, zn1, gsem, wsem):
    c = lax.axis_index("c")
    s = lax.axis_index("s")
    wid = s * NC + c
    zbufs = ((zs0, zn0), (zs1, zn1))
    ibufs = ((is0, in0), (is1, in1))

    def fire(sb, par):
        zsb, znb = zbufs[par]
        isb, inb = ibufs[par]
        pltpu.sync_copy(sidx_hbm.at[wid, sb], isb)
        pltpu.sync_copy(nidx_hbm.at[wid, sb], inb)
        for j in range(SBR2):
            pltpu.async_copy(ps_hbm.at[isb.at[j]],
                             zsb.at[pl.ds(j * CH2, CH2)], gsem)
            pltpu.async_copy(pn_hbm.at[inb.at[j]],
                             znb.at[pl.ds(j * CH2, CH2)], gsem)

    def drain_gathers(par):
        zsb, znb = zbufs[par]
        isb, inb = ibufs[par]
        for j in range(SBR2):
            pltpu.make_async_copy(ps_hbm.at[isb.at[j]],
                                  zsb.at[pl.ds(j * CH2, CH2)], gsem).wait()
            pltpu.make_async_copy(pn_hbm.at[inb.at[j]],
                                  znb.at[pl.ds(j * CH2, CH2)], gsem).wait()

    def fire_write(sb, par):
        zsb, znb = zbufs[par]
        ebase = wid * EPW + sb * _SBE2
        pltpu.async_copy(zsb, zs_out.at[pl.ds(ebase, _SBE2)], wsem)
        pltpu.async_copy(znb, zn_out.at[pl.ds(ebase, _SBE2)], wsem)

    def drain_write(par):
        zsb, znb = zbufs[par]
        pltpu.make_async_copy(zs_out.at[pl.ds(0, _SBE2)], zsb, wsem).wait()
        pltpu.make_async_copy(zn_out.at[pl.ds(0, _SBE2)], znb, wsem).wait()

    fire(0, 0)

    def body(k, carry):
        sb0 = 2 * k
        drain_gathers(0)

        @pl.when(k > 0)
        def _():
            drain_write(1)
        fire(sb0 + 1, 1)
        fire_write(sb0, 0)
        drain_gathers(1)
        drain_write(0)

        @pl.when(k < _NSB2 // 2 - 1)
        def _():
            fire(sb0 + 2, 0)
        fire_write(sb0 + 1, 1)
        return carry

    lax.fori_loop(0, _NSB2 // 2, body, 0)
    drain_write(1)


def _sc_pass2a(ps, pn, sidx3, nidx3):
    kfn = pl.kernel(
        _sc_pass2a_body,
        out_type=[
            jax.ShapeDtypeStruct((E, D), jnp.int32),
            jax.ShapeDtypeStruct((E, D), jnp.int32),
        ],
        mesh=_mesh(),
        scratch_types=[
            pltpu.VMEM((SBR2, CH2), jnp.int32),
            pltpu.VMEM((SBR2, CH2), jnp.int32),
            pltpu.VMEM((SBR2, CH2), jnp.int32),
            pltpu.VMEM((SBR2, CH2), jnp.int32),
            pltpu.VMEM((_SBE2, D), jnp.int32),
            pltpu.VMEM((_SBE2, D), jnp.int32),
            pltpu.VMEM((_SBE2, D), jnp.int32),
            pltpu.VMEM((_SBE2, D), jnp.int32),
            pltpu.SemaphoreType.DMA,
            pltpu.SemaphoreType.DMA,
        ],
    )
    return kfn(ps, pn, sidx3, nidx3)


def _sc_pass2c_body(msg_hbm, sidx_hbm, z128_hbm, part_out,
                    acc, idxs_v, mbuf, sem):
    c = lax.axis_index("c")
    s = lax.axis_index("s")
    wid = s * NC + c
    sbe = CH3 * SBR3
    nsb = EPW // sbe

    row0 = s * RPT
    pltpu.sync_copy(z128_hbm.at[pl.ds(row0, RPT)], acc.at[pl.ds(row0, RPT)])
    plsc.subcore_barrier()

    def body(sb, carry):
        ebase = wid * EPW + sb * sbe
        pltpu.sync_copy(sidx_hbm.at[wid, sb], idxs_v)
        g = pltpu.async_copy(msg_hbm.at[pl.ds(ebase, sbe)], mbuf, sem)
        g.wait()
        for j in range(SBR3):
            pltpu.sync_copy(mbuf.at[pl.ds(j * CH3, CH3)],
                            acc.at[idxs_v.at[j]], add=True)
        return carry

    lax.fori_loop(0, nsb, body, 0)
    plsc.subcore_barrier()
    sl = pl.ds(row0, RPT)
    pltpu.sync_copy(acc.at[sl], part_out.at[c, sl])


def _sc_pass2c(msg, sidx4, z128):
    kfn = pl.kernel(
        _sc_pass2c_body,
        out_type=jax.ShapeDtypeStruct((NC, NP, D), jnp.float32),
        mesh=_mesh(),
        scratch_types=[
            pltpu.VMEM_SHARED((NP, D), jnp.float32),
            pltpu.VMEM((SBR3, CH3), jnp.int32),
            pltpu.VMEM((SBR3 * CH3, D), jnp.float32),
            pltpu.SemaphoreType.DMA,
        ],
    )
    return kfn(msg, sidx4, z128)


def _pack2(x256):
    a = jax.lax.bitcast_convert_type(
        x256[:, :D].astype(jnp.bfloat16), jnp.uint16).astype(jnp.uint32)
    b = jax.lax.bitcast_convert_type(
        x256[:, D:].astype(jnp.bfloat16), jnp.uint16).astype(jnp.uint32)
    return jax.lax.bitcast_convert_type(a | (b << 16), jnp.int32)


def _unpack2(xi32):
    u = jax.lax.bitcast_convert_type(xi32, jnp.uint32)
    a = jax.lax.bitcast_convert_type(
        (u & jnp.uint32(0xFFFF)).astype(jnp.uint16), jnp.bfloat16)
    b = jax.lax.bitcast_convert_type(
        (u >> 16).astype(jnp.uint16), jnp.bfloat16)
    return jnp.concatenate(
        [a.astype(jnp.float32), b.astype(jnp.float32)], axis=-1)


def _tc_proj_body(atom_ref, wst_ref, wnt_ref, ps_ref, pn_ref):
    a = atom_ref[...]
    ps_ref[...] = _pack2(jax.lax.dot(a, wst_ref[...],
                                     precision=jax.lax.Precision.HIGHEST,
                                     preferred_element_type=jnp.float32))
    pn_ref[...] = _pack2(jax.lax.dot(a, wnt_ref[...],
                                     precision=jax.lax.Precision.HIGHEST,
                                     preferred_element_type=jnp.float32))


_PJ_BN = 2048


def _tc_proj(atom_p, wst, wnt):
    return pl.pallas_call(
        _tc_proj_body,
        grid=(NP // _PJ_BN,),
        in_specs=[
            pl.BlockSpec((_PJ_BN, D), lambda i: (i, 0)),
            pl.BlockSpec((D, F), lambda i: (0, 0)),
            pl.BlockSpec((D, F), lambda i: (0, 0)),
        ],
        out_specs=[
            pl.BlockSpec((_PJ_BN, D), lambda i: (i, 0)),
            pl.BlockSpec((_PJ_BN, D), lambda i: (i, 0)),
        ],
        out_shape=[
            jax.ShapeDtypeStruct((NP, D), jnp.int32),
            jax.ShapeDtypeStruct((NP, D), jnp.int32),
        ],
    )(atom_p, wst, wnt)


_MOM_BE = 8000


def _tc_mom_body(nbr_ref, m_ref):
    @pl.when(pl.program_id(0) == 0)
    def _init():
        m_ref[...] = jnp.zeros_like(m_ref)

    blk = nbr_ref[...]
    m_ref[...] += jax.lax.dot(blk.T, blk,
                              precision=jax.lax.Precision.HIGHEST,
                              preferred_element_type=jnp.float32)


def _tc_mom(nbr32):
    return pl.pallas_call(
        _tc_mom_body,
        grid=(E // _MOM_BE,),
        in_specs=[pl.BlockSpec((_MOM_BE, 32), lambda i: (i, 0))],
        out_specs=pl.BlockSpec((32, 32), lambda i: (0, 0)),
        out_shape=jax.ShapeDtypeStruct((32, 32), jnp.float32),
    )(nbr32)


_ST_BN = 1024


def _tc_stats_body(ps_ref, pn_ref, vp_ref, usp_ref, unp_ref, m32_ref,
                   w_ref, vecs_ref, out_ref, acc_ref):
    i = pl.program_id(0)
    hp = jax.lax.Precision.HIGHEST
    dot = functools.partial(jax.lax.dot, precision=hp,
                            preferred_element_type=jnp.float32)
    w = w_ref[...]
    wn = w[:, D:F]
    we = w[:, F:]
    b = vecs_ref[0]

    @pl.when(i == 0)
    def _init():
        acc_ref[...] = jnp.zeros_like(acc_ref)

    ps = _unpack2(ps_ref[...])
    pn = _unpack2(pn_ref[...])
    v = vp_ref[0] + vp_ref[1]
    us_w = usp_ref[0] + usp_ref[1]
    un_w = unp_ref[0] + unp_ref[1]
    u_s = us_w[:, :DE]
    u_n = un_w[:, :DE]
    cnt_s = us_w[:, DE]
    cnt_n = un_w[:, DE]

    s_pn = dot(v, wn.T)
    tq_s = dot(u_s, we.T) + cnt_s[:, None] * b
    tq_n = dot(u_n, we.T) + cnt_n[:, None] * b
    s1_k = dot(cnt_s[None, :], ps)[0] + dot(cnt_n[None, :], pn)[0]
    s2_k = (dot(cnt_s[None, :], ps * ps)[0]
            + dot(cnt_n[None, :], pn * pn)[0]
            + 2.0 * (jnp.sum(ps * s_pn, axis=0)
                     + jnp.sum(ps * tq_s, axis=0)
                     + jnp.sum(pn * tq_n, axis=0)))
    acc_ref[0, :] += s1_k
    acc_ref[1, :] += s2_k

    @pl.when(i == pl.num_programs(0) - 1)
    def _fin():
        bn1w = vecs_ref[1]
        bn1b = vecs_ref[2]
        m32 = m32_ref[...]
        csum = m32[DE, :DE]
        mm = m32[:DE, :DE]
        e_f = jnp.float32(E)
        qc = dot(csum[None, :], we.T)[0]
        sum_q = qc + e_f * b
        wem = dot(we, mm)
        sum_q2 = jnp.sum(wem * we, axis=1) + 2.0 * b * qc + e_f * b * b
        s1 = acc_ref[0, :] + sum_q
        s2 = acc_ref[1, :] + sum_q2
        mu = s1 / e_f
        var = s2 / e_f - mu * mu
        g1 = bn1w * jax.lax.rsqrt(var + jnp.float32(1e-5))
        c1 = bn1b - mu * g1
        out_ref[0, :] = g1
        out_ref[1, :] = c1


def _tc_stats(ps, pn, vp, usp, unp, m32, w, vecs):
    return pl.pallas_call(
        _tc_stats_body,
        grid=(NP // _ST_BN,),
        in_specs=[
            pl.BlockSpec((_ST_BN, D), lambda i: (i, 0)),
            pl.BlockSpec((_ST_BN, D), lambda i: (i, 0)),
            pl.BlockSpec((2, _ST_BN, D), lambda i: (0, i, 0)),
            pl.BlockSpec((2, _ST_BN, D), lambda i: (0, i, 0)),
            pl.BlockSpec((2, _ST_BN, D), lambda i: (0, i, 0)),
            pl.BlockSpec((32, 32), lambda i: (0, 0)),
            pl.BlockSpec((F, 272), lambda i: (0, 0)),
            pl.BlockSpec((4, F), lambda i: (0, 0)),
        ],
        out_specs=pl.BlockSpec((2, F), lambda i: (0, 0)),
        out_shape=jax.ShapeDtypeStruct((2, F), jnp.float32),
        scratch_shapes=[pltpu.VMEM((2, F), jnp.float32)],
    )(ps, pn, vp, usp, unp, m32, w, vecs)


_MSG_BE = 3200


def _tc_msg_body(zs_ref, zn_ref, nbr_ref, wet32_ref, g1c1_ref, msg_ref):
    q = jax.lax.dot(nbr_ref[...], wet32_ref[...],
                    precision=jax.lax.Precision.HIGHEST,
                    preferred_element_type=jnp.float32)
    zh = (_unpack2(zs_ref[...]) + _unpack2(zn_ref[...]) + q) \
        * g1c1_ref[0] + g1c1_ref[1]
    f = zh[:, :D]
    c = zh[:, D:]
    msg_ref[...] = jax.nn.sigmoid(f) * jax.nn.softplus(c)


def _tc_msg(zs, zn, nbr32, wet32, g1c1):
    return pl.pallas_call(
        _tc_msg_body,
        grid=(E // _MSG_BE,),
        in_specs=[
            pl.BlockSpec((_MSG_BE, D), lambda i: (i, 0)),
            pl.BlockSpec((_MSG_BE, D), lambda i: (i, 0)),
            pl.BlockSpec((_MSG_BE, 32), lambda i: (i, 0)),
            pl.BlockSpec((32, F), lambda i: (0, 0)),
            pl.BlockSpec((2, F), lambda i: (0, 0)),
        ],
        out_specs=pl.BlockSpec((_MSG_BE, D), lambda i: (i, 0)),
        out_shape=jax.ShapeDtypeStruct((E, D), jnp.float32),
    )(zs, zn, nbr32, wet32, g1c1)


def _tc_final_body(part_ref, atom_ref, vecs_ref, out_ref):
    p = part_ref[0, :N] + part_ref[1, :N]
    mu = jnp.mean(p, axis=0, keepdims=True)
    var = jnp.mean(p * p, axis=0, keepdims=True) - mu * mu
    g = vecs_ref[0] * jax.lax.rsqrt(var[0] + jnp.float32(1e-5))
    bnp = (p - mu[0]) * g + vecs_ref[1]
    out_ref[...] = jax.nn.softplus(atom_ref[...] + bnp)


def _tc_final(part, atom, vecs2):
    return pl.pallas_call(
        _tc_final_body,
        out_shape=jax.ShapeDtypeStruct((N, D), jnp.float32),
    )(part, atom, vecs2)


def kernel(atom_in_fea, nbr_fea, self_fea_idx, nbr_fea_idx, W, b,
           bn1_w, bn1_b, bn2_w, bn2_b):
    atom = atom_in_fea.astype(jnp.float32)
    nbr = nbr_fea.astype(jnp.float32)
    s32 = self_fea_idx.astype(jnp.int32)
    n32 = nbr_fea_idx.astype(jnp.int32)
    sidx_a = s32.reshape(NW, EPW // (CH1 * SBR1), SBR1, CH1)
    nidx_a = n32.reshape(NW, EPW // (CH1 * SBR1), SBR1, CH1)
    sidx_b = s32.reshape(NW, EPW // (CH1B * SBR1B), SBR1B, CH1B)
    nidx_b = n32.reshape(NW, EPW // (CH1B * SBR1B), SBR1B, CH1B)
    sidx_2 = s32.reshape(NW, EPW // (CH2 * SBR2), SBR2, CH2)
    nidx_2 = n32.reshape(NW, EPW // (CH2 * SBR2), SBR2, CH2)

    nbr32 = jnp.concatenate(
        [nbr, jnp.ones((E, 1), jnp.float32), jnp.zeros((E, 32 - DE - 1), jnp.float32)],
        axis=1)
    nbr128 = jnp.concatenate(
        [nbr32, jnp.zeros((E, D - 32), jnp.float32)], axis=1)

    wst = W[:, :D].T
    wnt = W[:, D:F].T
    wet32 = jnp.concatenate(
        [W[:, F:].T, b[None, :], jnp.zeros((32 - DE - 1, F), jnp.float32)],
        axis=0)
    vecs = jnp.stack([b, bn1_w, bn1_b, jnp.zeros_like(b)])
    vecs2 = jnp.stack([bn2_w, bn2_b])

    z128 = jnp.zeros((NP, D), jnp.float32)
    atom_p = jnp.concatenate([atom, jnp.zeros((NP - N, D), jnp.float32)], 0)

    ps, pn = _tc_proj(atom_p, wst, wnt)
    m32 = _tc_mom(nbr32)
    vp = _sc_pass1a(atom_p, sidx_a, nidx_a, z128)
    usp = _sc_pass1b(nbr128, sidx_b, z128)
    unp = _sc_pass1b(nbr128, nidx_b, z128)
    g1c1 = _tc_stats(ps, pn, vp, usp, unp, m32, W, vecs)
    zs, zn = _sc_pass2a(ps, pn, sidx_2, nidx_2)
    msg = _tc_msg(zs, zn, nbr32, wet32, g1c1)
    part = _sc_pass2c(msg, sidx_a, z128)
    return _tc_final(part, atom, vecs2)


# speedup vs baseline: 1.0460x; 1.0460x over previous
import functools

import jax
import jax.numpy as jnp
from jax import lax
from jax.experimental import pallas as pl
from jax.experimental.pallas import tpu as pltpu
from jax.experimental.pallas import tpu_sc as plsc

N = 10000
NP = 10240
E = 320000
D = 128
DE = 16
F = 256
NC = 2
NS = 16
NW = NC * NS
EPW = E // NW
RPT = NP // NS

CH1, SBR1 = 40, 5
CH1B, SBR1B = 40, 5
CH2, SBR2 = 80, 5
CH3, SBR3 = 40, 5


def _mesh():
    return plsc.VectorSubcoreMesh(core_axis_name="c", subcore_axis_name="s")


def _sc_pass1a_body(atom_hbm, sidx_hbm, nidx_hbm, z128_hbm, v_out,
                    v_acc, idxs_v, idxn_v, rows_v, sem, ssem):
    c = lax.axis_index("c")
    s = lax.axis_index("s")
    wid = s * NC + c
    nsb = EPW // (CH1 * SBR1)

    row0 = s * RPT
    pltpu.sync_copy(z128_hbm.at[pl.ds(row0, RPT)], v_acc.at[pl.ds(row0, RPT)])
    plsc.subcore_barrier()

    def body(sb, carry):
        pltpu.sync_copy(sidx_hbm.at[wid, sb], idxs_v)
        pltpu.sync_copy(nidx_hbm.at[wid, sb], idxn_v)
        gets = [
            pltpu.async_copy(atom_hbm.at[idxn_v.at[j]],
                             rows_v.at[pl.ds(j * CH1, CH1)], sem)
            for j in range(SBR1)
        ]
        puts = []
        for j in range(SBR1):
            gets[j].wait()
            puts.append(pltpu.async_copy(rows_v.at[pl.ds(j * CH1, CH1)],
                                         v_acc.at[idxs_v.at[j]], ssem,
                                         add=True))
        for p in puts:
            p.wait()
        return carry

    lax.fori_loop(0, nsb, body, 0)
    plsc.subcore_barrier()
    sl = pl.ds(row0, RPT)
    pltpu.sync_copy(v_acc.at[sl], v_out.at[c, sl])


def _sc_pass1a(atom, sidx4, nidx4, z128):
    kfn = pl.kernel(
        _sc_pass1a_body,
        out_type=jax.ShapeDtypeStruct((NC, NP, D), jnp.float32),
        mesh=_mesh(),
        scratch_types=[
            pltpu.VMEM_SHARED((NP, D), jnp.float32),
            pltpu.VMEM((SBR1, CH1), jnp.int32),
            pltpu.VMEM((SBR1, CH1), jnp.int32),
            pltpu.VMEM((SBR1 * CH1, D), jnp.float32),
            pltpu.SemaphoreType.DMA,
            pltpu.SemaphoreType.DMA,
        ],
    )
    return kfn(atom, sidx4, nidx4, z128)


def _sc_pass1b_body(nbr128_hbm, idx_hbm, z128_hbm, u_out,
                    u_acc, idx_v, wide_v, sem):
    c = lax.axis_index("c")
    s = lax.axis_index("s")
    wid = s * NC + c
    sbe = CH1B * SBR1B
    nsb = EPW // sbe

    row0 = s * RPT
    pltpu.sync_copy(z128_hbm.at[pl.ds(row0, RPT)], u_acc.at[pl.ds(row0, RPT)])
    plsc.subcore_barrier()

    def body(sb, carry):
        ebase = wid * EPW + sb * sbe
        pltpu.sync_copy(idx_hbm.at[wid, sb], idx_v)
        g = pltpu.async_copy(nbr128_hbm.at[pl.ds(ebase, sbe)], wide_v, sem)
        g.wait()
        puts = [
            pltpu.async_copy(wide_v.at[pl.ds(j * CH1B, CH1B)],
                             u_acc.at[idx_v.at[j]], sem, add=True)
            for j in range(SBR1B)
        ]
        for p in puts:
            p.wait()
        return carry

    lax.fori_loop(0, nsb, body, 0)
    plsc.subcore_barrier()
    sl = pl.ds(row0, RPT)
    pltpu.sync_copy(u_acc.at[sl], u_out.at[c, sl])


def _sc_pass1b(nbr128, idx4, z128):
    kfn = pl.kernel(
        _sc_pass1b_body,
        out_type=jax.ShapeDtypeStruct((NC, NP, D), jnp.float32),
        mesh=_mesh(),
        scratch_types=[
            pltpu.VMEM_SHARED((NP, D), jnp.float32),
            pltpu.VMEM((SBR1B, CH1B), jnp.int32),
            pltpu.VMEM((SBR1B * CH1B, D), jnp.float32),
            pltpu.SemaphoreType.DMA,
        ],
    )
    return kfn(nbr128, idx4, z128)


def _sc_pass2a_body(ps_hbm, pn_hbm, sidx_hbm, nidx_hbm, zs_out, zn_out,
                    idxs_v, idxn_v, zsbuf, znbuf, sem, sem2):
    c = lax.axis_index("c")
    s = lax.axis_index("s")
    wid = s * NC + c
    sbe = CH2 * SBR2
    nsb = EPW // sbe

    def body(sb, carry):
        ebase = wid * EPW + sb * sbe
        pltpu.sync_copy(sidx_hbm.at[wid, sb], idxs_v)
        pltpu.sync_copy(nidx_hbm.at[wid, sb], idxn_v)
        gets = [
            pltpu.async_copy(ps_hbm.at[idxs_v.at[j]],
                             zsbuf.at[pl.ds(j * CH2, CH2)], sem)
            for j in range(SBR2)
        ] + [
            pltpu.async_copy(pn_hbm.at[idxn_v.at[j]],
                             znbuf.at[pl.ds(j * CH2, CH2)], sem2)
            for j in range(SBR2)
        ]
        for g in gets:
            g.wait()
        w1 = pltpu.async_copy(zsbuf, zs_out.at[pl.ds(ebase, sbe)], sem)
        w2 = pltpu.async_copy(znbuf, zn_out.at[pl.ds(ebase, sbe)], sem2)
        w1.wait()
        w2.wait()
        return carry

    lax.fori_loop(0, nsb, body, 0)


def _sc_pass2a(ps, pn, sidx4, nidx4):
    kfn = pl.kernel(
        _sc_pass2a_body,
        out_type=[
            jax.ShapeDtypeStruct((E, D), jnp.int32),
            jax.ShapeDtypeStruct((E, D), jnp.int32),
        ],
        mesh=_mesh(),
        scratch_types=[
            pltpu.VMEM((SBR2, CH2), jnp.int32),
            pltpu.VMEM((SBR2, CH2), jnp.int32),
            pltpu.VMEM((SBR2 * CH2, D), jnp.int32),
            pltpu.VMEM((SBR2 * CH2, D), jnp.int32),
            pltpu.SemaphoreType.DMA,
            pltpu.SemaphoreType.DMA,
        ],
    )
    return kfn(ps, pn, sidx4, nidx4)


def _sc_pass2c_body(msg_hbm, sidx_hbm, z128_hbm, part_out,
                    acc, idxs_v, mbuf, sem):
    c = lax.axis_index("c")
    s = lax.axis_index("s")
    wid = s * NC + c
    sbe = CH3 * SBR3
    nsb = EPW // sbe

    row0 = s * RPT
    pltpu.sync_copy(z128_hbm.at[pl.ds(row0, RPT)], acc.at[pl.ds(row0, RPT)])
    plsc.subcore_barrier()

    def body(sb, carry):
        ebase = wid * EPW + sb * sbe
        pltpu.sync_copy(sidx_hbm.at[wid, sb], idxs_v)
        g = pltpu.async_copy(msg_hbm.at[pl.ds(ebase, sbe)], mbuf, sem)
        g.wait()
        puts = [
            pltpu.async_copy(mbuf.at[pl.ds(j * CH3, CH3)],
                             acc.at[idxs_v.at[j]], sem, add=True)
            for j in range(SBR3)
        ]
        for p in puts:
            p.wait()
        return carry

    lax.fori_loop(0, nsb, body, 0)
    plsc.subcore_barrier()
    sl = pl.ds(row0, RPT)
    pltpu.sync_copy(acc.at[sl], part_out.at[c, sl])


def _sc_pass2c(msg, sidx4, z128):
    kfn = pl.kernel(
        _sc_pass2c_body,
        out_type=jax.ShapeDtypeStruct((NC, NP, D), jnp.float32),
        mesh=_mesh(),
        scratch_types=[
            pltpu.VMEM_SHARED((NP, D), jnp.float32),
            pltpu.VMEM((SBR3, CH3), jnp.int32),
            pltpu.VMEM((SBR3 * CH3, D), jnp.float32),
            pltpu.SemaphoreType.DMA,
        ],
    )
    return kfn(msg, sidx4, z128)


def _pack2(x256):
    a = jax.lax.bitcast_convert_type(
        x256[:, :D].astype(jnp.bfloat16), jnp.uint16).astype(jnp.uint32)
    b = jax.lax.bitcast_convert_type(
        x256[:, D:].astype(jnp.bfloat16), jnp.uint16).astype(jnp.uint32)
    return jax.lax.bitcast_convert_type(a | (b << 16), jnp.int32)


def _unpack2(xi32):
    u = jax.lax.bitcast_convert_type(xi32, jnp.uint32)
    a = jax.lax.bitcast_convert_type(
        (u & jnp.uint32(0xFFFF)).astype(jnp.uint16), jnp.bfloat16)
    b = jax.lax.bitcast_convert_type(
        (u >> 16).astype(jnp.uint16), jnp.bfloat16)
    return jnp.concatenate(
        [a.astype(jnp.float32), b.astype(jnp.float32)], axis=-1)


def _tc_proj_body(atom_ref, wst_ref, wnt_ref, ps_ref, pn_ref):
    a = atom_ref[...]
    ps_ref[...] = _pack2(jax.lax.dot(a, wst_ref[...],
                                     precision=jax.lax.Precision.HIGHEST,
                                     preferred_element_type=jnp.float32))
    pn_ref[...] = _pack2(jax.lax.dot(a, wnt_ref[...],
                                     precision=jax.lax.Precision.HIGHEST,
                                     preferred_element_type=jnp.float32))


_PJ_BN = 2048


def _tc_proj(atom_p, wst, wnt):
    return pl.pallas_call(
        _tc_proj_body,
        grid=(NP // _PJ_BN,),
        in_specs=[
            pl.BlockSpec((_PJ_BN, D), lambda i: (i, 0)),
            pl.BlockSpec((D, F), lambda i: (0, 0)),
            pl.BlockSpec((D, F), lambda i: (0, 0)),
        ],
        out_specs=[
            pl.BlockSpec((_PJ_BN, D), lambda i: (i, 0)),
            pl.BlockSpec((_PJ_BN, D), lambda i: (i, 0)),
        ],
        out_shape=[
            jax.ShapeDtypeStruct((NP, D), jnp.int32),
            jax.ShapeDtypeStruct((NP, D), jnp.int32),
        ],
    )(atom_p, wst, wnt)


_MOM_BE = 8000


def _tc_mom_body(nbr_ref, m_ref):
    @pl.when(pl.program_id(0) == 0)
    def _init():
        m_ref[...] = jnp.zeros_like(m_ref)

    blk = nbr_ref[...]
    m_ref[...] += jax.lax.dot(blk.T, blk,
                              precision=jax.lax.Precision.HIGHEST,
                              preferred_element_type=jnp.float32)


def _tc_mom(nbr32):
    return pl.pallas_call(
        _tc_mom_body,
        grid=(E // _MOM_BE,),
        in_specs=[pl.BlockSpec((_MOM_BE, 32), lambda i: (i, 0))],
        out_specs=pl.BlockSpec((32, 32), lambda i: (0, 0)),
        out_shape=jax.ShapeDtypeStruct((32, 32), jnp.float32),
    )(nbr32)


_ST_BN = 1024


def _tc_stats_body(ps_ref, pn_ref, vp_ref, usp_ref, unp_ref, m32_ref,
                   w_ref, vecs_ref, out_ref, acc_ref):
    i = pl.program_id(0)
    hp = jax.lax.Precision.HIGHEST
    dot = functools.partial(jax.lax.dot, precision=hp,
                            preferred_element_type=jnp.float32)
    w = w_ref[...]
    wn = w[:, D:F]
    we = w[:, F:]
    b = vecs_ref[0]

    @pl.when(i == 0)
    def _init():
        acc_ref[...] = jnp.zeros_like(acc_ref)

    ps = _unpack2(ps_ref[...])
    pn = _unpack2(pn_ref[...])
    v = vp_ref[0] + vp_ref[1]
    us_w = usp_ref[0] + usp_ref[1]
    un_w = unp_ref[0] + unp_ref[1]
    u_s = us_w[:, :DE]
    u_n = un_w[:, :DE]
    cnt_s = us_w[:, DE]
    cnt_n = un_w[:, DE]

    s_pn = dot(v, wn.T)
    tq_s = dot(u_s, we.T) + cnt_s[:, None] * b
    tq_n = dot(u_n, we.T) + cnt_n[:, None] * b
    s1_k = dot(cnt_s[None, :], ps)[0] + dot(cnt_n[None, :], pn)[0]
    s2_k = (dot(cnt_s[None, :], ps * ps)[0]
            + dot(cnt_n[None, :], pn * pn)[0]
            + 2.0 * (jnp.sum(ps * s_pn, axis=0)
                     + jnp.sum(ps * tq_s, axis=0)
                     + jnp.sum(pn * tq_n, axis=0)))
    acc_ref[0, :] += s1_k
    acc_ref[1, :] += s2_k

    @pl.when(i == pl.num_programs(0) - 1)
    def _fin():
        bn1w = vecs_ref[1]
        bn1b = vecs_ref[2]
        m32 = m32_ref[...]
        csum = m32[DE, :DE]
        mm = m32[:DE, :DE]
        e_f = jnp.float32(E)
        qc = dot(csum[None, :], we.T)[0]
        sum_q = qc + e_f * b
        wem = dot(we, mm)
        sum_q2 = jnp.sum(wem * we, axis=1) + 2.0 * b * qc + e_f * b * b
        s1 = acc_ref[0, :] + sum_q
        s2 = acc_ref[1, :] + sum_q2
        mu = s1 / e_f
        var = s2 / e_f - mu * mu
        g1 = bn1w * jax.lax.rsqrt(var + jnp.float32(1e-5))
        c1 = bn1b - mu * g1
        out_ref[0, :] = g1
        out_ref[1, :] = c1


def _tc_stats(ps, pn, vp, usp, unp, m32, w, vecs):
    return pl.pallas_call(
        _tc_stats_body,
        grid=(NP // _ST_BN,),
        in_specs=[
            pl.BlockSpec((_ST_BN, D), lambda i: (i, 0)),
            pl.BlockSpec((_ST_BN, D), lambda i: (i, 0)),
            pl.BlockSpec((2, _ST_BN, D), lambda i: (0, i, 0)),
            pl.BlockSpec((2, _ST_BN, D), lambda i: (0, i, 0)),
            pl.BlockSpec((2, _ST_BN, D), lambda i: (0, i, 0)),
            pl.BlockSpec((32, 32), lambda i: (0, 0)),
            pl.BlockSpec((F, 272), lambda i: (0, 0)),
            pl.BlockSpec((4, F), lambda i: (0, 0)),
        ],
        out_specs=pl.BlockSpec((2, F), lambda i: (0, 0)),
        out_shape=jax.ShapeDtypeStruct((2, F), jnp.float32),
        scratch_shapes=[pltpu.VMEM((2, F), jnp.float32)],
    )(ps, pn, vp, usp, unp, m32, w, vecs)


_MSG_BE = 3200


def _tc_msg_body(zs_ref, zn_ref, nbr_ref, wet32_ref, g1c1_ref, msg_ref):
    q = jax.lax.dot(nbr_ref[...], wet32_ref[...],
                    precision=jax.lax.Precision.HIGHEST,
                    preferred_element_type=jnp.float32)
    zh = (_unpack2(zs_ref[...]) + _unpack2(zn_ref[...]) + q) \
        * g1c1_ref[0] + g1c1_ref[1]
    f = zh[:, :D]
    c = zh[:, D:]
    msg_ref[...] = jax.nn.sigmoid(f) * jax.nn.softplus(c)


def _tc_msg(zs, zn, nbr32, wet32, g1c1):
    return pl.pallas_call(
        _tc_msg_body,
        grid=(E // _MSG_BE,),
        in_specs=[
            pl.BlockSpec((_MSG_BE, D), lambda i: (i, 0)),
            pl.BlockSpec((_MSG_BE, D), lambda i: (i, 0)),
            pl.BlockSpec((_MSG_BE, 32), lambda i: (i, 0)),
            pl.BlockSpec((32, F), lambda i: (0, 0)),
            pl.BlockSpec((2, F), lambda i: (0, 0)),
        ],
        out_specs=pl.BlockSpec((_MSG_BE, D), lambda i: (i, 0)),
        out_shape=jax.ShapeDtypeStruct((E, D), jnp.float32),
    )(zs, zn, nbr32, wet32, g1c1)


def _tc_final_body(part_ref, atom_ref, vecs_ref, out_ref):
    p = part_ref[0, :N] + part_ref[1, :N]
    mu = jnp.mean(p, axis=0, keepdims=True)
    var = jnp.mean(p * p, axis=0, keepdims=True) - mu * mu
    g = vecs_ref[0] * jax.lax.rsqrt(var[0] + jnp.float32(1e-5))
    bnp = (p - mu[0]) * g + vecs_ref[1]
    out_ref[...] = jax.nn.softplus(atom_ref[...] + bnp)


def _tc_final(part, atom, vecs2):
    return pl.pallas_call(
        _tc_final_body,
        out_shape=jax.ShapeDtypeStruct((N, D), jnp.float32),
    )(part, atom, vecs2)


def kernel(atom_in_fea, nbr_fea, self_fea_idx, nbr_fea_idx, W, b,
           bn1_w, bn1_b, bn2_w, bn2_b):
    atom = atom_in_fea.astype(jnp.float32)
    nbr = nbr_fea.astype(jnp.float32)
    s32 = self_fea_idx.astype(jnp.int32)
    n32 = nbr_fea_idx.astype(jnp.int32)
    sidx_a = s32.reshape(NW, EPW // (CH1 * SBR1), SBR1, CH1)
    nidx_a = n32.reshape(NW, EPW // (CH1 * SBR1), SBR1, CH1)
    sidx_b = s32.reshape(NW, EPW // (CH1B * SBR1B), SBR1B, CH1B)
    nidx_b = n32.reshape(NW, EPW // (CH1B * SBR1B), SBR1B, CH1B)
    sidx_2 = s32.reshape(NW, EPW // (CH2 * SBR2), SBR2, CH2)
    nidx_2 = n32.reshape(NW, EPW // (CH2 * SBR2), SBR2, CH2)

    nbr32 = jnp.concatenate(
        [nbr, jnp.ones((E, 1), jnp.float32), jnp.zeros((E, 32 - DE - 1), jnp.float32)],
        axis=1)
    nbr128 = jnp.concatenate(
        [nbr32, jnp.zeros((E, D - 32), jnp.float32)], axis=1)

    wst = W[:, :D].T
    wnt = W[:, D:F].T
    wet32 = jnp.concatenate(
        [W[:, F:].T, b[None, :], jnp.zeros((32 - DE - 1, F), jnp.float32)],
        axis=0)
    vecs = jnp.stack([b, bn1_w, bn1_b, jnp.zeros_like(b)])
    vecs2 = jnp.stack([bn2_w, bn2_b])

    z128 = jnp.zeros((NP, D), jnp.float32)
    atom_p = jnp.concatenate([atom, jnp.zeros((NP - N, D), jnp.float32)], 0)

    ps, pn = _tc_proj(atom_p, wst, wnt)
    m32 = _tc_mom(nbr32)
    vp = _sc_pass1a(atom_p, sidx_a, nidx_a, z128)
    usp = _sc_pass1b(nbr128, sidx_b, z128)
    unp = _sc_pass1b(nbr128, nidx_b, z128)
    g1c1 = _tc_stats(ps, pn, vp, usp, unp, m32, W, vecs)
    zs, zn = _sc_pass2a(ps, pn, sidx_2, nidx_2)
    msg = _tc_msg(zs, zn, nbr32, wet32, g1c1)
    part = _sc_pass2c(msg, sidx_a, z128)
    return _tc_final(part, atom, vecs2)


# speedup vs baseline: 1.0494x; 1.0032x over previous
import functools

import jax
import jax.numpy as jnp
from jax import lax
from jax.experimental import pallas as pl
from jax.experimental.pallas import tpu as pltpu
from jax.experimental.pallas import tpu_sc as plsc

N = 10000
NP = 10240
E = 320000
D = 128
DE = 16
F = 256
NC = 2
NS = 16
NW = NC * NS
EPW = E // NW
RPT = NP // NS

CH1, SBR1 = 40, 5
CH1B, SBR1B = 40, 5
CH2, SBR2 = 80, 5
CH3, SBR3 = 40, 5


def _mesh():
    return plsc.VectorSubcoreMesh(core_axis_name="c", subcore_axis_name="s")


def _sc_pass1a_body(atom_hbm, sidx_hbm, nidx_hbm, z128_hbm, v_out,
                    v_acc, idxs_v, idxn_v, rows_v, sem, ssem):
    c = lax.axis_index("c")
    s = lax.axis_index("s")
    wid = s * NC + c
    nsb = EPW // (CH1 * SBR1)

    row0 = s * RPT
    pltpu.sync_copy(z128_hbm.at[pl.ds(row0, RPT)], v_acc.at[pl.ds(row0, RPT)])
    plsc.subcore_barrier()

    def body(sb, carry):
        pltpu.sync_copy(sidx_hbm.at[wid, sb], idxs_v)
        pltpu.sync_copy(nidx_hbm.at[wid, sb], idxn_v)
        gets = [
            pltpu.async_copy(atom_hbm.at[idxn_v.at[j]],
                             rows_v.at[pl.ds(j * CH1, CH1)], sem)
            for j in range(SBR1)
        ]
        puts = []
        for j in range(SBR1):
            gets[j].wait()
            puts.append(pltpu.async_copy(rows_v.at[pl.ds(j * CH1, CH1)],
                                         v_acc.at[idxs_v.at[j]], ssem,
                                         add=True))
        for p in puts:
            p.wait()
        return carry

    lax.fori_loop(0, nsb, body, 0)
    plsc.subcore_barrier()
    sl = pl.ds(row0, RPT)
    pltpu.sync_copy(v_acc.at[sl], v_out.at[c, sl])


def _sc_pass1a(atom, sidx4, nidx4, z128):
    kfn = pl.kernel(
        _sc_pass1a_body,
        out_type=jax.ShapeDtypeStruct((NC, NP, D), jnp.float32),
        mesh=_mesh(),
        scratch_types=[
            pltpu.VMEM_SHARED((NP, D), jnp.float32),
            pltpu.VMEM((SBR1, CH1), jnp.int32),
            pltpu.VMEM((SBR1, CH1), jnp.int32),
            pltpu.VMEM((SBR1 * CH1, D), jnp.float32),
            pltpu.SemaphoreType.DMA,
            pltpu.SemaphoreType.DMA,
        ],
    )
    return kfn(atom, sidx4, nidx4, z128)


def _sc_pass1b_body(nbr128_hbm, idx_hbm, z128_hbm, u_out,
                    u_acc, idx_v, wide_v, sem):
    c = lax.axis_index("c")
    s = lax.axis_index("s")
    wid = s * NC + c
    sbe = CH1B * SBR1B
    nsb = EPW // sbe

    row0 = s * RPT
    pltpu.sync_copy(z128_hbm.at[pl.ds(row0, RPT)], u_acc.at[pl.ds(row0, RPT)])
    plsc.subcore_barrier()

    def body(sb, carry):
        ebase = wid * EPW + sb * sbe
        pltpu.sync_copy(idx_hbm.at[wid, sb], idx_v)
        g = pltpu.async_copy(nbr128_hbm.at[pl.ds(ebase, sbe)], wide_v, sem)
        g.wait()
        puts = [
            pltpu.async_copy(wide_v.at[pl.ds(j * CH1B, CH1B)],
                             u_acc.at[idx_v.at[j]], sem, add=True)
            for j in range(SBR1B)
        ]
        for p in puts:
            p.wait()
        return carry

    lax.fori_loop(0, nsb, body, 0)
    plsc.subcore_barrier()
    sl = pl.ds(row0, RPT)
    pltpu.sync_copy(u_acc.at[sl], u_out.at[c, sl])


def _sc_pass1b(nbr128, idx4, z128):
    kfn = pl.kernel(
        _sc_pass1b_body,
        out_type=jax.ShapeDtypeStruct((NC, NP, D), jnp.float32),
        mesh=_mesh(),
        scratch_types=[
            pltpu.VMEM_SHARED((NP, D), jnp.float32),
            pltpu.VMEM((SBR1B, CH1B), jnp.int32),
            pltpu.VMEM((SBR1B * CH1B, D), jnp.float32),
            pltpu.SemaphoreType.DMA,
        ],
    )
    return kfn(nbr128, idx4, z128)


def _sc_pass2a_body(ps_hbm, pn_hbm, sidx_hbm, nidx_hbm, zs_out, zn_out,
                    idxs_v, idxn_v, zsbuf, znbuf, sem, sem2):
    c = lax.axis_index("c")
    s = lax.axis_index("s")
    wid = s * NC + c
    sbe = CH2 * SBR2
    nsb = EPW // sbe

    def body(sb, carry):
        ebase = wid * EPW + sb * sbe
        pltpu.sync_copy(sidx_hbm.at[wid, sb], idxs_v)
        pltpu.sync_copy(nidx_hbm.at[wid, sb], idxn_v)
        gets = [
            pltpu.async_copy(ps_hbm.at[idxs_v.at[j]],
                             zsbuf.at[pl.ds(j * CH2, CH2)], sem)
            for j in range(SBR2)
        ] + [
            pltpu.async_copy(pn_hbm.at[idxn_v.at[j]],
                             znbuf.at[pl.ds(j * CH2, CH2)], sem2)
            for j in range(SBR2)
        ]
        for g in gets:
            g.wait()
        w1 = pltpu.async_copy(zsbuf, zs_out.at[pl.ds(ebase, sbe)], sem)
        w2 = pltpu.async_copy(znbuf, zn_out.at[pl.ds(ebase, sbe)], sem2)
        w1.wait()
        w2.wait()
        return carry

    lax.fori_loop(0, nsb, body, 0)


def _sc_pass2a(ps, pn, sidx4, nidx4):
    kfn = pl.kernel(
        _sc_pass2a_body,
        out_type=[
            jax.ShapeDtypeStruct((E, D), jnp.int32),
            jax.ShapeDtypeStruct((E, D), jnp.int32),
        ],
        mesh=_mesh(),
        scratch_types=[
            pltpu.VMEM((SBR2, CH2), jnp.int32),
            pltpu.VMEM((SBR2, CH2), jnp.int32),
            pltpu.VMEM((SBR2 * CH2, D), jnp.int32),
            pltpu.VMEM((SBR2 * CH2, D), jnp.int32),
            pltpu.SemaphoreType.DMA,
            pltpu.SemaphoreType.DMA,
        ],
    )
    return kfn(ps, pn, sidx4, nidx4)


def _sc_pass2c_body(msg_hbm, sidx_hbm, z128_hbm, part_out,
                    acc, idxs_v, mbuf, sem):
    c = lax.axis_index("c")
    s = lax.axis_index("s")
    wid = s * NC + c
    sbe = CH3 * SBR3
    nsb = EPW // sbe

    row0 = s * RPT
    pltpu.sync_copy(z128_hbm.at[pl.ds(row0, RPT)], acc.at[pl.ds(row0, RPT)])
    plsc.subcore_barrier()

    def body(sb, carry):
        ebase = wid * EPW + sb * sbe
        pltpu.sync_copy(sidx_hbm.at[wid, sb], idxs_v)
        g = pltpu.async_copy(msg_hbm.at[pl.ds(ebase, sbe)], mbuf, sem)
        g.wait()
        puts = [
            pltpu.async_copy(mbuf.at[pl.ds(j * CH3, CH3)],
                             acc.at[idxs_v.at[j]], sem, add=True)
            for j in range(SBR3)
        ]
        for p in puts:
            p.wait()
        return carry

    lax.fori_loop(0, nsb, body, 0)
    plsc.subcore_barrier()
    sl = pl.ds(row0, RPT)
    pltpu.sync_copy(acc.at[sl], part_out.at[c, sl])


def _sc_pass2c(msg, sidx4, z128):
    kfn = pl.kernel(
        _sc_pass2c_body,
        out_type=jax.ShapeDtypeStruct((NC, NP, D), jnp.float32),
        mesh=_mesh(),
        scratch_types=[
            pltpu.VMEM_SHARED((NP, D), jnp.float32),
            pltpu.VMEM((SBR3, CH3), jnp.int32),
            pltpu.VMEM((SBR3 * CH3, D), jnp.float32),
            pltpu.SemaphoreType.DMA,
        ],
    )
    return kfn(msg, sidx4, z128)


def _pack2(x256):
    a = jax.lax.bitcast_convert_type(
        x256[:, :D].astype(jnp.bfloat16), jnp.uint16).astype(jnp.uint32)
    b = jax.lax.bitcast_convert_type(
        x256[:, D:].astype(jnp.bfloat16), jnp.uint16).astype(jnp.uint32)
    return jax.lax.bitcast_convert_type(a | (b << 16), jnp.int32)


def _unpack2(xi32):
    u = jax.lax.bitcast_convert_type(xi32, jnp.uint32)
    a = jax.lax.bitcast_convert_type(
        (u & jnp.uint32(0xFFFF)).astype(jnp.uint16), jnp.bfloat16)
    b = jax.lax.bitcast_convert_type(
        (u >> 16).astype(jnp.uint16), jnp.bfloat16)
    return jnp.concatenate(
        [a.astype(jnp.float32), b.astype(jnp.float32)], axis=-1)


def _tc_proj_body(atom_ref, wst_ref, wnt_ref, ps_ref, pn_ref):
    a = atom_ref[...]
    ps_ref[...] = _pack2(jax.lax.dot(a, wst_ref[...],
                                     precision=jax.lax.Precision.HIGHEST,
                                     preferred_element_type=jnp.float32))
    pn_ref[...] = _pack2(jax.lax.dot(a, wnt_ref[...],
                                     precision=jax.lax.Precision.HIGHEST,
                                     preferred_element_type=jnp.float32))


_PJ_BN = 2048


def _tc_proj(atom_p, wst, wnt):
    return pl.pallas_call(
        _tc_proj_body,
        grid=(NP // _PJ_BN,),
        in_specs=[
            pl.BlockSpec((_PJ_BN, D), lambda i: (i, 0)),
            pl.BlockSpec((D, F), lambda i: (0, 0)),
            pl.BlockSpec((D, F), lambda i: (0, 0)),
        ],
        out_specs=[
            pl.BlockSpec((_PJ_BN, D), lambda i: (i, 0)),
            pl.BlockSpec((_PJ_BN, D), lambda i: (i, 0)),
        ],
        out_shape=[
            jax.ShapeDtypeStruct((NP, D), jnp.int32),
            jax.ShapeDtypeStruct((NP, D), jnp.int32),
        ],
    )(atom_p, wst, wnt)


_MOM_BE = 8000


def _tc_mom_body(nbr_ref, m_ref):
    @pl.when(pl.program_id(0) == 0)
    def _init():
        m_ref[...] = jnp.zeros_like(m_ref)

    blk = nbr_ref[...]
    m_ref[...] += jax.lax.dot(blk.T, blk,
                              precision=jax.lax.Precision.HIGHEST,
                              preferred_element_type=jnp.float32)


def _tc_mom(nbr32):
    return pl.pallas_call(
        _tc_mom_body,
        grid=(E // _MOM_BE,),
        in_specs=[pl.BlockSpec((_MOM_BE, 32), lambda i: (i, 0))],
        out_specs=pl.BlockSpec((32, 32), lambda i: (0, 0)),
        out_shape=jax.ShapeDtypeStruct((32, 32), jnp.float32),
    )(nbr32)


_ST_BN = 1024


def _tc_stats_body(ps_ref, pn_ref, vp_ref, usp_ref, unp_ref, m32_ref,
                   w_ref, vecs_ref, out_ref, acc_ref):
    i = pl.program_id(0)
    hp = jax.lax.Precision.HIGHEST
    dot = functools.partial(jax.lax.dot, precision=hp,
                            preferred_element_type=jnp.float32)
    w = w_ref[...]
    wn = w[:, D:F]
    we = w[:, F:]
    b = vecs_ref[0]

    @pl.when(i == 0)
    def _init():
        acc_ref[...] = jnp.zeros_like(acc_ref)

    ps = _unpack2(ps_ref[...])
    pn = _unpack2(pn_ref[...])
    v = vp_ref[0] + vp_ref[1]
    us_w = usp_ref[0] + usp_ref[1]
    un_w = unp_ref[0] + unp_ref[1]
    u_s = us_w[:, :DE]
    u_n = un_w[:, :DE]
    cnt_s = us_w[:, DE]
    cnt_n = un_w[:, DE]

    s_pn = dot(v, wn.T)
    tq_s = dot(u_s, we.T) + cnt_s[:, None] * b
    tq_n = dot(u_n, we.T) + cnt_n[:, None] * b
    s1_k = dot(cnt_s[None, :], ps)[0] + dot(cnt_n[None, :], pn)[0]
    s2_k = (dot(cnt_s[None, :], ps * ps)[0]
            + dot(cnt_n[None, :], pn * pn)[0]
            + 2.0 * (jnp.sum(ps * s_pn, axis=0)
                     + jnp.sum(ps * tq_s, axis=0)
                     + jnp.sum(pn * tq_n, axis=0)))
    acc_ref[0, :] += s1_k
    acc_ref[1, :] += s2_k

    @pl.when(i == pl.num_programs(0) - 1)
    def _fin():
        bn1w = vecs_ref[1]
        bn1b = vecs_ref[2]
        m32 = m32_ref[...]
        csum = m32[DE, :DE]
        mm = m32[:DE, :DE]
        e_f = jnp.float32(E)
        qc = dot(csum[None, :], we.T)[0]
        sum_q = qc + e_f * b
        wem = dot(we, mm)
        sum_q2 = jnp.sum(wem * we, axis=1) + 2.0 * b * qc + e_f * b * b
        s1 = acc_ref[0, :] + sum_q
        s2 = acc_ref[1, :] + sum_q2
        mu = s1 / e_f
        var = s2 / e_f - mu * mu
        g1 = bn1w * jax.lax.rsqrt(var + jnp.float32(1e-5))
        c1 = bn1b - mu * g1
        out_ref[0, :] = g1
        out_ref[1, :] = c1


def _tc_stats(ps, pn, vp, usp, unp, m32, w, vecs):
    return pl.pallas_call(
        _tc_stats_body,
        grid=(NP // _ST_BN,),
        in_specs=[
            pl.BlockSpec((_ST_BN, D), lambda i: (i, 0)),
            pl.BlockSpec((_ST_BN, D), lambda i: (i, 0)),
            pl.BlockSpec((2, _ST_BN, D), lambda i: (0, i, 0)),
            pl.BlockSpec((2, _ST_BN, D), lambda i: (0, i, 0)),
            pl.BlockSpec((2, _ST_BN, D), lambda i: (0, i, 0)),
            pl.BlockSpec((32, 32), lambda i: (0, 0)),
            pl.BlockSpec((F, 272), lambda i: (0, 0)),
            pl.BlockSpec((4, F), lambda i: (0, 0)),
        ],
        out_specs=pl.BlockSpec((2, F), lambda i: (0, 0)),
        out_shape=jax.ShapeDtypeStruct((2, F), jnp.float32),
        scratch_shapes=[pltpu.VMEM((2, F), jnp.float32)],
    )(ps, pn, vp, usp, unp, m32, w, vecs)


_MSG_BE = 6400


def _tc_msg_body(zs_ref, zn_ref, nbr_ref, wet32_ref, g1c1_ref, msg_ref):
    q = jax.lax.dot(nbr_ref[...], wet32_ref[...],
                    precision=jax.lax.Precision.HIGHEST,
                    preferred_element_type=jnp.float32)
    zh = (_unpack2(zs_ref[...]) + _unpack2(zn_ref[...]) + q) \
        * g1c1_ref[0] + g1c1_ref[1]
    f = zh[:, :D]
    c = zh[:, D:]
    msg_ref[...] = jax.nn.sigmoid(f) * jax.nn.softplus(c)


def _tc_msg(zs, zn, nbr32, wet32, g1c1):
    return pl.pallas_call(
        _tc_msg_body,
        grid=(E // _MSG_BE,),
        in_specs=[
            pl.BlockSpec((_MSG_BE, D), lambda i: (i, 0)),
            pl.BlockSpec((_MSG_BE, D), lambda i: (i, 0)),
            pl.BlockSpec((_MSG_BE, 32), lambda i: (i, 0)),
            pl.BlockSpec((32, F), lambda i: (0, 0)),
            pl.BlockSpec((2, F), lambda i: (0, 0)),
        ],
        out_specs=pl.BlockSpec((_MSG_BE, D), lambda i: (i, 0)),
        out_shape=jax.ShapeDtypeStruct((E, D), jnp.float32),
    )(zs, zn, nbr32, wet32, g1c1)


def _tc_final_body(part_ref, atom_ref, vecs_ref, out_ref):
    p = part_ref[0, :N] + part_ref[1, :N]
    mu = jnp.mean(p, axis=0, keepdims=True)
    var = jnp.mean(p * p, axis=0, keepdims=True) - mu * mu
    g = vecs_ref[0] * jax.lax.rsqrt(var[0] + jnp.float32(1e-5))
    bnp = (p - mu[0]) * g + vecs_ref[1]
    out_ref[...] = jax.nn.softplus(atom_ref[...] + bnp)


def _tc_final(part, atom, vecs2):
    return pl.pallas_call(
        _tc_final_body,
        out_shape=jax.ShapeDtypeStruct((N, D), jnp.float32),
    )(part, atom, vecs2)


def kernel(atom_in_fea, nbr_fea, self_fea_idx, nbr_fea_idx, W, b,
           bn1_w, bn1_b, bn2_w, bn2_b):
    atom = atom_in_fea.astype(jnp.float32)
    nbr = nbr_fea.astype(jnp.float32)
    s32 = self_fea_idx.astype(jnp.int32)
    n32 = nbr_fea_idx.astype(jnp.int32)
    sidx_a = s32.reshape(NW, EPW // (CH1 * SBR1), SBR1, CH1)
    nidx_a = n32.reshape(NW, EPW // (CH1 * SBR1), SBR1, CH1)
    sidx_b = s32.reshape(NW, EPW // (CH1B * SBR1B), SBR1B, CH1B)
    nidx_b = n32.reshape(NW, EPW // (CH1B * SBR1B), SBR1B, CH1B)
    sidx_2 = s32.reshape(NW, EPW // (CH2 * SBR2), SBR2, CH2)
    nidx_2 = n32.reshape(NW, EPW // (CH2 * SBR2), SBR2, CH2)

    nbr32 = jnp.concatenate(
        [nbr, jnp.ones((E, 1), jnp.float32), jnp.zeros((E, 32 - DE - 1), jnp.float32)],
        axis=1)
    nbr128 = jnp.concatenate(
        [nbr32, jnp.zeros((E, D - 32), jnp.float32)], axis=1)

    wst = W[:, :D].T
    wnt = W[:, D:F].T
    wet32 = jnp.concatenate(
        [W[:, F:].T, b[None, :], jnp.zeros((32 - DE - 1, F), jnp.float32)],
        axis=0)
    vecs = jnp.stack([b, bn1_w, bn1_b, jnp.zeros_like(b)])
    vecs2 = jnp.stack([bn2_w, bn2_b])

    z128 = jnp.zeros((NP, D), jnp.float32)
    atom_p = jnp.concatenate([atom, jnp.zeros((NP - N, D), jnp.float32)], 0)

    ps, pn = _tc_proj(atom_p, wst, wnt)
    m32 = _tc_mom(nbr32)
    vp = _sc_pass1a(atom_p, sidx_a, nidx_a, z128)
    usp = _sc_pass1b(nbr128, sidx_b, z128)
    unp = _sc_pass1b(nbr128, nidx_b, z128)
    g1c1 = _tc_stats(ps, pn, vp, usp, unp, m32, W, vecs)
    zs, zn = _sc_pass2a(ps, pn, sidx_2, nidx_2)
    msg = _tc_msg(zs, zn, nbr32, wet32, g1c1)
    part = _sc_pass2c(msg, sidx_a, z128)
    return _tc_final(part, atom, vecs2)


# speedup vs baseline: 1.0716x; 1.0211x over previous
import functools

import jax
import jax.numpy as jnp
from jax import lax
from jax.experimental import pallas as pl
from jax.experimental.pallas import tpu as pltpu
from jax.experimental.pallas import tpu_sc as plsc

N = 10000
NP = 10240
E = 320000
D = 128
DE = 16
F = 256
NC = 2
NS = 16
NW = NC * NS
EPW = E // NW
RPT = NP // NS

CH1, SBR1 = 40, 5
CH1B, SBR1B = 40, 5
CH2, SBR2 = 80, 5
CH3, SBR3 = 40, 5


def _mesh():
    return plsc.VectorSubcoreMesh(core_axis_name="c", subcore_axis_name="s")


def _sc_pass1a_body(atom_hbm, sidx_hbm, nidx_hbm, z128_hbm, v_out,
                    v_acc, idxs_v, idxn_v, rows_v, sem, ssem):
    c = lax.axis_index("c")
    s = lax.axis_index("s")
    wid = s * NC + c
    nsb = EPW // (CH1 * SBR1)

    row0 = s * RPT
    pltpu.sync_copy(z128_hbm.at[pl.ds(row0, RPT)], v_acc.at[pl.ds(row0, RPT)])
    plsc.subcore_barrier()

    def body(sb, carry):
        pltpu.sync_copy(sidx_hbm.at[wid, sb], idxs_v)
        pltpu.sync_copy(nidx_hbm.at[wid, sb], idxn_v)
        gets = [
            pltpu.async_copy(atom_hbm.at[idxn_v.at[j]],
                             rows_v.at[pl.ds(j * CH1, CH1)], sem)
            for j in range(SBR1)
        ]
        puts = []
        for j in range(SBR1):
            gets[j].wait()
            puts.append(pltpu.async_copy(rows_v.at[pl.ds(j * CH1, CH1)],
                                         v_acc.at[idxs_v.at[j]], ssem,
                                         add=True))
        for p in puts:
            p.wait()
        return carry

    lax.fori_loop(0, nsb, body, 0)
    plsc.subcore_barrier()
    sl = pl.ds(row0, RPT)
    pltpu.sync_copy(v_acc.at[sl], v_out.at[c, sl])


def _sc_pass1a(atom, sidx4, nidx4, z128):
    kfn = pl.kernel(
        _sc_pass1a_body,
        out_type=jax.ShapeDtypeStruct((NC, NP, D), jnp.float32),
        mesh=_mesh(),
        scratch_types=[
            pltpu.VMEM_SHARED((NP, D), jnp.float32),
            pltpu.VMEM((SBR1, CH1), jnp.int32),
            pltpu.VMEM((SBR1, CH1), jnp.int32),
            pltpu.VMEM((SBR1 * CH1, D), jnp.float32),
            pltpu.SemaphoreType.DMA,
            pltpu.SemaphoreType.DMA,
        ],
    )
    return kfn(atom, sidx4, nidx4, z128)


def _sc_pass1b_body(nbr128_hbm, idx_hbm, z128_hbm, u_out,
                    u_acc, idx_v, wide_v, sem):
    c = lax.axis_index("c")
    s = lax.axis_index("s")
    wid = s * NC + c
    sbe = CH1B * SBR1B
    nsb = EPW // sbe

    row0 = s * RPT
    pltpu.sync_copy(z128_hbm.at[pl.ds(row0, RPT)], u_acc.at[pl.ds(row0, RPT)])
    plsc.subcore_barrier()

    def body(sb, carry):
        ebase = wid * EPW + sb * sbe
        pltpu.sync_copy(idx_hbm.at[wid, sb], idx_v)
        g = pltpu.async_copy(nbr128_hbm.at[pl.ds(ebase, sbe)], wide_v, sem)
        g.wait()
        puts = [
            pltpu.async_copy(wide_v.at[pl.ds(j * CH1B, CH1B)],
                             u_acc.at[idx_v.at[j]], sem, add=True)
            for j in range(SBR1B)
        ]
        for p in puts:
            p.wait()
        return carry

    lax.fori_loop(0, nsb, body, 0)
    plsc.subcore_barrier()
    sl = pl.ds(row0, RPT)
    pltpu.sync_copy(u_acc.at[sl], u_out.at[c, sl])


def _sc_pass1b(nbr128, idx4, z128):
    kfn = pl.kernel(
        _sc_pass1b_body,
        out_type=jax.ShapeDtypeStruct((NC, NP, D), jnp.float32),
        mesh=_mesh(),
        scratch_types=[
            pltpu.VMEM_SHARED((NP, D), jnp.float32),
            pltpu.VMEM((SBR1B, CH1B), jnp.int32),
            pltpu.VMEM((SBR1B * CH1B, D), jnp.float32),
            pltpu.SemaphoreType.DMA,
        ],
    )
    return kfn(nbr128, idx4, z128)


def _sc_pass2a_body(ps_hbm, pn_hbm, sidx_hbm, nidx_hbm, zs_out, zn_out,
                    is0, in0, is1, in1, zsbuf, znbuf, sem, sem2, isem):
    c = lax.axis_index("c")
    s = lax.axis_index("s")
    wid = s * NC + c
    sbe = CH2 * SBR2
    nsb = EPW // sbe
    ibufs = ((is0, in0), (is1, in1))

    pltpu.sync_copy(sidx_hbm.at[wid, 0], is0)
    pltpu.sync_copy(nidx_hbm.at[wid, 0], in0)
    for sb in range(nsb):
        isb, inb = ibufs[sb % 2]
        ebase = wid * EPW + sb * sbe
        gets = [
            pltpu.async_copy(ps_hbm.at[isb.at[j]],
                             zsbuf.at[pl.ds(j * CH2, CH2)], sem)
            for j in range(SBR2)
        ] + [
            pltpu.async_copy(pn_hbm.at[inb.at[j]],
                             znbuf.at[pl.ds(j * CH2, CH2)], sem2)
            for j in range(SBR2)
        ]
        ipf = []
        if sb + 1 < nsb:
            nisb, ninb = ibufs[(sb + 1) % 2]
            ipf = [pltpu.async_copy(sidx_hbm.at[wid, sb + 1], nisb, isem),
                   pltpu.async_copy(nidx_hbm.at[wid, sb + 1], ninb, isem)]
        for g in gets:
            g.wait()
        w1 = pltpu.async_copy(zsbuf, zs_out.at[pl.ds(ebase, sbe)], sem)
        w2 = pltpu.async_copy(znbuf, zn_out.at[pl.ds(ebase, sbe)], sem2)
        for g in ipf:
            g.wait()
        w1.wait()
        w2.wait()


def _sc_pass2a(ps, pn, sidx4, nidx4):
    kfn = pl.kernel(
        _sc_pass2a_body,
        out_type=[
            jax.ShapeDtypeStruct((E, D), jnp.int32),
            jax.ShapeDtypeStruct((E, D), jnp.int32),
        ],
        mesh=_mesh(),
        scratch_types=[
            pltpu.VMEM((SBR2, CH2), jnp.int32),
            pltpu.VMEM((SBR2, CH2), jnp.int32),
            pltpu.VMEM((SBR2, CH2), jnp.int32),
            pltpu.VMEM((SBR2, CH2), jnp.int32),
            pltpu.VMEM((SBR2 * CH2, D), jnp.int32),
            pltpu.VMEM((SBR2 * CH2, D), jnp.int32),
            pltpu.SemaphoreType.DMA,
            pltpu.SemaphoreType.DMA,
            pltpu.SemaphoreType.DMA,
        ],
    )
    return kfn(ps, pn, sidx4, nidx4)


def _sc_pass2c_body(msg_hbm, sidx_hbm, z128_hbm, part_out,
                    acc, idxs_v, mbuf, sem):
    c = lax.axis_index("c")
    s = lax.axis_index("s")
    wid = s * NC + c
    sbe = CH3 * SBR3
    nsb = EPW // sbe

    row0 = s * RPT
    pltpu.sync_copy(z128_hbm.at[pl.ds(row0, RPT)], acc.at[pl.ds(row0, RPT)])
    plsc.subcore_barrier()

    def body(sb, carry):
        ebase = wid * EPW + sb * sbe
        pltpu.sync_copy(sidx_hbm.at[wid, sb], idxs_v)
        g = pltpu.async_copy(msg_hbm.at[pl.ds(ebase, sbe)], mbuf, sem)
        g.wait()
        puts = [
            pltpu.async_copy(mbuf.at[pl.ds(j * CH3, CH3)],
                             acc.at[idxs_v.at[j]], sem, add=True)
            for j in range(SBR3)
        ]
        for p in puts:
            p.wait()
        return carry

    lax.fori_loop(0, nsb, body, 0)
    plsc.subcore_barrier()
    sl = pl.ds(row0, RPT)
    pltpu.sync_copy(acc.at[sl], part_out.at[c, sl])


def _sc_pass2c(msg, sidx4, z128):
    kfn = pl.kernel(
        _sc_pass2c_body,
        out_type=jax.ShapeDtypeStruct((NC, NP, D), jnp.float32),
        mesh=_mesh(),
        scratch_types=[
            pltpu.VMEM_SHARED((NP, D), jnp.float32),
            pltpu.VMEM((SBR3, CH3), jnp.int32),
            pltpu.VMEM((SBR3 * CH3, D), jnp.float32),
            pltpu.SemaphoreType.DMA,
        ],
    )
    return kfn(msg, sidx4, z128)


def _pack2(x256):
    a = jax.lax.bitcast_convert_type(
        x256[:, :D].astype(jnp.bfloat16), jnp.uint16).astype(jnp.uint32)
    b = jax.lax.bitcast_convert_type(
        x256[:, D:].astype(jnp.bfloat16), jnp.uint16).astype(jnp.uint32)
    return jax.lax.bitcast_convert_type(a | (b << 16), jnp.int32)


def _unpack2(xi32):
    u = jax.lax.bitcast_convert_type(xi32, jnp.uint32)
    a = jax.lax.bitcast_convert_type(
        (u & jnp.uint32(0xFFFF)).astype(jnp.uint16), jnp.bfloat16)
    b = jax.lax.bitcast_convert_type(
        (u >> 16).astype(jnp.uint16), jnp.bfloat16)
    return jnp.concatenate(
        [a.astype(jnp.float32), b.astype(jnp.float32)], axis=-1)


def _tc_proj_body(atom_ref, wst_ref, wnt_ref, ps_ref, pn_ref):
    a = atom_ref[...]
    ps_ref[...] = _pack2(jax.lax.dot(a, wst_ref[...],
                                     precision=jax.lax.Precision.HIGHEST,
                                     preferred_element_type=jnp.float32))
    pn_ref[...] = _pack2(jax.lax.dot(a, wnt_ref[...],
                                     precision=jax.lax.Precision.HIGHEST,
                                     preferred_element_type=jnp.float32))


_PJ_BN = 2048


def _tc_proj(atom_p, wst, wnt):
    return pl.pallas_call(
        _tc_proj_body,
        grid=(NP // _PJ_BN,),
        in_specs=[
            pl.BlockSpec((_PJ_BN, D), lambda i: (i, 0)),
            pl.BlockSpec((D, F), lambda i: (0, 0)),
            pl.BlockSpec((D, F), lambda i: (0, 0)),
        ],
        out_specs=[
            pl.BlockSpec((_PJ_BN, D), lambda i: (i, 0)),
            pl.BlockSpec((_PJ_BN, D), lambda i: (i, 0)),
        ],
        out_shape=[
            jax.ShapeDtypeStruct((NP, D), jnp.int32),
            jax.ShapeDtypeStruct((NP, D), jnp.int32),
        ],
    )(atom_p, wst, wnt)


_MOM_BE = 8000


def _tc_mom_body(nbr_ref, m_ref):
    @pl.when(pl.program_id(0) == 0)
    def _init():
        m_ref[...] = jnp.zeros_like(m_ref)

    blk = nbr_ref[...]
    m_ref[...] += jax.lax.dot(blk.T, blk,
                              precision=jax.lax.Precision.HIGHEST,
                              preferred_element_type=jnp.float32)


def _tc_mom(nbr32):
    return pl.pallas_call(
        _tc_mom_body,
        grid=(E // _MOM_BE,),
        in_specs=[pl.BlockSpec((_MOM_BE, 32), lambda i: (i, 0))],
        out_specs=pl.BlockSpec((32, 32), lambda i: (0, 0)),
        out_shape=jax.ShapeDtypeStruct((32, 32), jnp.float32),
    )(nbr32)


_ST_BN = 1024


def _tc_stats_body(ps_ref, pn_ref, vp_ref, usp_ref, unp_ref, m32_ref,
                   w_ref, vecs_ref, out_ref, acc_ref):
    i = pl.program_id(0)
    hp = jax.lax.Precision.HIGHEST
    dot = functools.partial(jax.lax.dot, precision=hp,
                            preferred_element_type=jnp.float32)
    w = w_ref[...]
    wn = w[:, D:F]
    we = w[:, F:]
    b = vecs_ref[0]

    @pl.when(i == 0)
    def _init():
        acc_ref[...] = jnp.zeros_like(acc_ref)

    ps = _unpack2(ps_ref[...])
    pn = _unpack2(pn_ref[...])
    v = vp_ref[0] + vp_ref[1]
    us_w = usp_ref[0] + usp_ref[1]
    un_w = unp_ref[0] + unp_ref[1]
    u_s = us_w[:, :DE]
    u_n = un_w[:, :DE]
    cnt_s = us_w[:, DE]
    cnt_n = un_w[:, DE]

    s_pn = dot(v, wn.T)
    tq_s = dot(u_s, we.T) + cnt_s[:, None] * b
    tq_n = dot(u_n, we.T) + cnt_n[:, None] * b
    s1_k = dot(cnt_s[None, :], ps)[0] + dot(cnt_n[None, :], pn)[0]
    s2_k = (dot(cnt_s[None, :], ps * ps)[0]
            + dot(cnt_n[None, :], pn * pn)[0]
            + 2.0 * (jnp.sum(ps * s_pn, axis=0)
                     + jnp.sum(ps * tq_s, axis=0)
                     + jnp.sum(pn * tq_n, axis=0)))
    acc_ref[0, :] += s1_k
    acc_ref[1, :] += s2_k

    @pl.when(i == pl.num_programs(0) - 1)
    def _fin():
        bn1w = vecs_ref[1]
        bn1b = vecs_ref[2]
        m32 = m32_ref[...]
        csum = m32[DE, :DE]
        mm = m32[:DE, :DE]
        e_f = jnp.float32(E)
        qc = dot(csum[None, :], we.T)[0]
        sum_q = qc + e_f * b
        wem = dot(we, mm)
        sum_q2 = jnp.sum(wem * we, axis=1) + 2.0 * b * qc + e_f * b * b
        s1 = acc_ref[0, :] + sum_q
        s2 = acc_ref[1, :] + sum_q2
        mu = s1 / e_f
        var = s2 / e_f - mu * mu
        g1 = bn1w * jax.lax.rsqrt(var + jnp.float32(1e-5))
        c1 = bn1b - mu * g1
        out_ref[0, :] = g1
        out_ref[1, :] = c1


def _tc_stats(ps, pn, vp, usp, unp, m32, w, vecs):
    return pl.pallas_call(
        _tc_stats_body,
        grid=(NP // _ST_BN,),
        in_specs=[
            pl.BlockSpec((_ST_BN, D), lambda i: (i, 0)),
            pl.BlockSpec((_ST_BN, D), lambda i: (i, 0)),
            pl.BlockSpec((2, _ST_BN, D), lambda i: (0, i, 0)),
            pl.BlockSpec((2, _ST_BN, D), lambda i: (0, i, 0)),
            pl.BlockSpec((2, _ST_BN, D), lambda i: (0, i, 0)),
            pl.BlockSpec((32, 32), lambda i: (0, 0)),
            pl.BlockSpec((F, 272), lambda i: (0, 0)),
            pl.BlockSpec((4, F), lambda i: (0, 0)),
        ],
        out_specs=pl.BlockSpec((2, F), lambda i: (0, 0)),
        out_shape=jax.ShapeDtypeStruct((2, F), jnp.float32),
        scratch_shapes=[pltpu.VMEM((2, F), jnp.float32)],
    )(ps, pn, vp, usp, unp, m32, w, vecs)


_MSG_BE = 6400


def _tc_msg_body(zs_ref, zn_ref, nbr_ref, wet32_ref, g1c1_ref, msg_ref):
    q = jax.lax.dot(nbr_ref[...], wet32_ref[...],
                    precision=jax.lax.Precision.HIGHEST,
                    preferred_element_type=jnp.float32)
    zh = (_unpack2(zs_ref[...]) + _unpack2(zn_ref[...]) + q) \
        * g1c1_ref[0] + g1c1_ref[1]
    f = zh[:, :D]
    c = zh[:, D:]
    msg_ref[...] = jax.nn.sigmoid(f) * jax.nn.softplus(c)


def _tc_msg(zs, zn, nbr32, wet32, g1c1):
    return pl.pallas_call(
        _tc_msg_body,
        grid=(E // _MSG_BE,),
        in_specs=[
            pl.BlockSpec((_MSG_BE, D), lambda i: (i, 0)),
            pl.BlockSpec((_MSG_BE, D), lambda i: (i, 0)),
            pl.BlockSpec((_MSG_BE, 32), lambda i: (i, 0)),
            pl.BlockSpec((32, F), lambda i: (0, 0)),
            pl.BlockSpec((2, F), lambda i: (0, 0)),
        ],
        out_specs=pl.BlockSpec((_MSG_BE, D), lambda i: (i, 0)),
        out_shape=jax.ShapeDtypeStruct((E, D), jnp.float32),
    )(zs, zn, nbr32, wet32, g1c1)


def _tc_final_body(part_ref, atom_ref, vecs_ref, out_ref):
    p = part_ref[0, :N] + part_ref[1, :N]
    mu = jnp.mean(p, axis=0, keepdims=True)
    var = jnp.mean(p * p, axis=0, keepdims=True) - mu * mu
    g = vecs_ref[0] * jax.lax.rsqrt(var[0] + jnp.float32(1e-5))
    bnp = (p - mu[0]) * g + vecs_ref[1]
    out_ref[...] = jax.nn.softplus(atom_ref[...] + bnp)


def _tc_final(part, atom, vecs2):
    return pl.pallas_call(
        _tc_final_body,
        out_shape=jax.ShapeDtypeStruct((N, D), jnp.float32),
    )(part, atom, vecs2)


def kernel(atom_in_fea, nbr_fea, self_fea_idx, nbr_fea_idx, W, b,
           bn1_w, bn1_b, bn2_w, bn2_b):
    atom = atom_in_fea.astype(jnp.float32)
    nbr = nbr_fea.astype(jnp.float32)
    s32 = self_fea_idx.astype(jnp.int32)
    n32 = nbr_fea_idx.astype(jnp.int32)
    sidx_a = s32.reshape(NW, EPW // (CH1 * SBR1), SBR1, CH1)
    nidx_a = n32.reshape(NW, EPW // (CH1 * SBR1), SBR1, CH1)
    sidx_b = s32.reshape(NW, EPW // (CH1B * SBR1B), SBR1B, CH1B)
    nidx_b = n32.reshape(NW, EPW // (CH1B * SBR1B), SBR1B, CH1B)
    sidx_2 = s32.reshape(NW, EPW // (CH2 * SBR2), SBR2, CH2)
    nidx_2 = n32.reshape(NW, EPW // (CH2 * SBR2), SBR2, CH2)

    nbr32 = jnp.concatenate(
        [nbr, jnp.ones((E, 1), jnp.float32), jnp.zeros((E, 32 - DE - 1), jnp.float32)],
        axis=1)
    nbr128 = jnp.concatenate(
        [nbr32, jnp.zeros((E, D - 32), jnp.float32)], axis=1)

    wst = W[:, :D].T
    wnt = W[:, D:F].T
    wet32 = jnp.concatenate(
        [W[:, F:].T, b[None, :], jnp.zeros((32 - DE - 1, F), jnp.float32)],
        axis=0)
    vecs = jnp.stack([b, bn1_w, bn1_b, jnp.zeros_like(b)])
    vecs2 = jnp.stack([bn2_w, bn2_b])

    z128 = jnp.zeros((NP, D), jnp.float32)
    atom_p = jnp.concatenate([atom, jnp.zeros((NP - N, D), jnp.float32)], 0)

    ps, pn = _tc_proj(atom_p, wst, wnt)
    m32 = _tc_mom(nbr32)
    vp = _sc_pass1a(atom_p, sidx_a, nidx_a, z128)
    usp = _sc_pass1b(nbr128, sidx_b, z128)
    unp = _sc_pass1b(nbr128, nidx_b, z128)
    g1c1 = _tc_stats(ps, pn, vp, usp, unp, m32, W, vecs)
    zs, zn = _sc_pass2a(ps, pn, sidx_2, nidx_2)
    msg = _tc_msg(zs, zn, nbr32, wet32, g1c1)
    part = _sc_pass2c(msg, sidx_a, z128)
    return _tc_final(part, atom, vecs2)
